# baseline (speedup 1.0000x reference)
import functools

import jax
import jax.numpy as jnp
from jax import lax
from jax.experimental import pallas as pl
from jax.experimental.pallas import tpu as pltpu
from jax.experimental.pallas import tpu_sc as plsc

N = 50000
E = 800000
NTILE = 16
NROW = 3128
N_PAD = NROW * NTILE
CW = 32
CHUNK = 256
SUB = 128
NSUB = CHUNK // SUB


SROUND = 8


def _pad_edges(src, dst):
    per_sr = CHUNK * SROUND
    per_tile = -(-E // (NTILE * per_sr)) * per_sr
    e_pad = per_tile * NTILE
    pad = e_pad - E
    pad_src = (jnp.arange(pad, dtype=jnp.int32) * 97) % N
    pad_dst = jnp.arange(pad, dtype=jnp.int32) % (N_PAD - N) + N
    src = jnp.concatenate([src.astype(jnp.int32), pad_src]).reshape(e_pad // SUB, SUB)
    dst = jnp.concatenate([dst.astype(jnp.int32), pad_dst]).reshape(e_pad // SUB, SUB)
    return src, dst, per_tile // CHUNK


def _make_segsum(n_groups, rounds):
    n_pass = n_groups // 2
    mesh = plsc.VectorSubcoreMesh(core_axis_name="c", subcore_axis_name="s")

    def body(*refs):
        tables = refs[:n_groups]
        (src_hbm, dst_hbm, z_hbm, agg_hbm,
         src_v, dst_v, buf0, buf1, buf2, buf3, acc_sh,
         sg0, sg1, sg2, sg3, ss0, ss1, ss2, ss3) = refs[n_groups:]
        buf = (buf0, buf1, buf2, buf3)
        sg = (sg0, sg1, sg2, sg3)
        ss = (ss0, ss1, ss2, ss3)
        c = lax.axis_index("c")
        s = lax.axis_index("s")
        idxr = SROUND * NSUB
        nsr = rounds // SROUND

        for p in range(n_pass):
            for cv in range(2):
                g = 2 * p + cv
                tab = tables[g]

                @pl.when(c == cv)
                def _(tab=tab, g=g):
                    pltpu.sync_copy(z_hbm, acc_sh.at[pl.ds(s * NROW, NROW)])
                    plsc.subcore_barrier()

                    def fire_gather(b, r):
                        pltpu.async_copy(tab.at[src_v.at[r]], buf[b], sg[b])

                    def fire_scatter(b, r):
                        pltpu.async_copy(buf[b], acc_sh.at[dst_v.at[r]],
                                         ss[b], add=True)

                    def drain(sem, b):
                        pltpu.make_async_copy(
                            tab.at[pl.ds(0, SUB)], buf[b], sem).wait()

                    def sround(k, _):
                        r0 = s * (rounds * NSUB) + k * idxr
                        pltpu.sync_copy(src_hbm.at[pl.ds(r0, idxr)], src_v)
                        pltpu.sync_copy(dst_hbm.at[pl.ds(r0, idxr)], dst_v)
                        fire_gather(0, 0)
                        for r in range(1, idxr):
                            b = r % 4
                            if r >= 4:
                                drain(ss[b], b)
                            fire_gather(b, r)
                            pb = (r - 1) % 4
                            drain(sg[pb], pb)
                            fire_scatter(pb, r - 1)
                        last = (idxr - 1) % 4
                        drain(sg[last], last)
                        fire_scatter(last, idxr - 1)
                        for b in range(4):
                            drain(ss[b], b)
                        return 0

                    lax.fori_loop(0, nsr, sround, 0)
                    plsc.subcore_barrier()
                    pltpu.sync_copy(
                        acc_sh.at[pl.ds(s * NROW, NROW)],
                        agg_hbm.at[g].at[pl.ds(s * NROW, NROW)])
                    plsc.subcore_barrier()

    return pl.kernel(
        body,
        mesh=mesh,
        compiler_params=pltpu.CompilerParams(use_tc_tiling_on_sc=False),
        out_type=jax.ShapeDtypeStruct((n_groups, N_PAD, CW), jnp.float32),
        scratch_types=[
            pltpu.VMEM((SROUND * NSUB, SUB), jnp.int32),
            pltpu.VMEM((SROUND * NSUB, SUB), jnp.int32),
            pltpu.VMEM((SUB, CW), jnp.float32),
            pltpu.VMEM((SUB, CW), jnp.float32),
            pltpu.VMEM((SUB, CW), jnp.float32),
            pltpu.VMEM((SUB, CW), jnp.float32),
            pltpu.VMEM_SHARED((N_PAD, CW), jnp.float32),
        ] + [pltpu.SemaphoreType.DMA] * 8,
    )


def _sc_segsum(x, src2, dst2, rounds, zeros):
    c_dim = x.shape[1]
    n_groups = c_dim // CW
    tables = [x[:, g * CW:(g + 1) * CW] for g in range(n_groups)]
    fn = _make_segsum(n_groups, rounds)
    agg = fn(*tables, src2, dst2, zeros)
    return jnp.moveaxis(agg[:, :N, :], 0, 1).reshape(N, c_dim)


BLK = 2000
NB = N // BLK
G = 512


def _a_body(x_ref, a1_ref, a2_ref, w1_ref, w2_ref, b1_ref, b2_ref,
            h1_ref, h2_ref, s1_ref, s2_ref):
    i = pl.program_id(0)
    for a_ref, w_ref, b_ref, h_ref, s_ref in (
            (a1_ref, w1_ref, b1_ref, h1_ref, s1_ref),
            (a2_ref, w2_ref, b2_ref, h2_ref, s2_ref)):
        h = jnp.dot(x_ref[...] + a_ref[...], w_ref[...],
                    preferred_element_type=jnp.float32) + b_ref[...]
        h_ref[...] = h
        z = jnp.zeros((6, 256), jnp.float32)
        ps = jnp.concatenate(
            [jnp.sum(h, 0)[None], jnp.sum(h * h, 0)[None], z], axis=0)

        @pl.when(i == 0)
        def _():
            s_ref[...] = ps

        @pl.when(i > 0)
        def _():
            s_ref[...] += ps


def _conv_pair(x, agg1, agg2, p1, p2):
    din = x.shape[1]
    return pl.pallas_call(
        _a_body,
        grid=(NB,),
        in_specs=[
            pl.BlockSpec((BLK, din), lambda i: (i, 0)),
            pl.BlockSpec((BLK, din), lambda i: (i, 0)),
            pl.BlockSpec((BLK, din), lambda i: (i, 0)),
            pl.BlockSpec((din, 256), lambda i: (0, 0)),
            pl.BlockSpec((din, 256), lambda i: (0, 0)),
            pl.BlockSpec((1, 256), lambda i: (0, 0)),
            pl.BlockSpec((1, 256), lambda i: (0, 0)),
        ],
        out_specs=[
            pl.BlockSpec((BLK, 256), lambda i: (i, 0)),
            pl.BlockSpec((BLK, 256), lambda i: (i, 0)),
            pl.BlockSpec((8, 256), lambda i: (0, 0)),
            pl.BlockSpec((8, 256), lambda i: (0, 0)),
        ],
        out_shape=[
            jax.ShapeDtypeStruct((N, 256), jnp.float32),
            jax.ShapeDtypeStruct((N, 256), jnp.float32),
            jax.ShapeDtypeStruct((8, 256), jnp.float32),
            jax.ShapeDtypeStruct((8, 256), jnp.float32),
        ],
    )(x, agg1, agg2, p1['lin1']['W'], p2['lin1']['W'],
      p1['lin1']['b'][None], p2['lin1']['b'][None])


def _b_body(h1_ref, h2_ref, s1_ref, s2_ref, g1_ref, be1_ref, g2_ref, be2_ref,
            w21_ref, c21_ref, w22_ref, c22_ref, wma_ref, wmb_ref, bm1_ref,
            wm2_ref, bm2_ref, o_ref):
    ys = []
    for h_ref, s_ref, g_ref, be_ref, w_ref, c_ref in (
            (h1_ref, s1_ref, g1_ref, be1_ref, w21_ref, c21_ref),
            (h2_ref, s2_ref, g2_ref, be2_ref, w22_ref, c22_ref)):
        mu = s_ref[0:1, :] / N
        var = s_ref[1:2, :] / N - mu * mu
        inv = jax.lax.rsqrt(var + 1e-5) * g_ref[...]
        hn = jax.nn.relu((h_ref[...] - mu) * inv + be_ref[...])
        ys.append(jax.nn.relu(
            jnp.dot(hn, w_ref[...], preferred_element_type=jnp.float32)
            + c_ref[...]))
    t = jax.nn.relu(
        jnp.dot(ys[0], wma_ref[...], preferred_element_type=jnp.float32)
        + jnp.dot(ys[1], wmb_ref[...], preferred_element_type=jnp.float32)
        + bm1_ref[...])
    o_ref[...] = jnp.dot(t, wm2_ref[...],
                         preferred_element_type=jnp.float32) + bm2_ref[...]


def _bn_mlp(h1, h2, s1, s2, p1, p2, mlp):
    row = pl.BlockSpec((BLK, 256), lambda i: (i, 0))
    w = pl.BlockSpec((256, 256), lambda i: (0, 0))
    b = pl.BlockSpec((1, 256), lambda i: (0, 0))
    s = pl.BlockSpec((8, 256), lambda i: (0, 0))
    wm = mlp['lin1']['W']
    return pl.pallas_call(
        _b_body,
        grid=(NB,),
        in_specs=[row, row, s, s, b, b, b, b, w, b, w, b, w, w, b, w, b],
        out_specs=row,
        out_shape=jax.ShapeDtypeStruct((N, 256), jnp.float32),
    )(h1, h2, s1, s2,
      p1['gamma'][None], p1['beta'][None], p2['gamma'][None], p2['beta'][None],
      p1['lin2']['W'], p1['lin2']['b'][None],
      p2['lin2']['W'], p2['lin2']['b'][None],
      wm[:256], wm[256:], mlp['lin1']['b'][None],
      mlp['lin2']['W'], mlp['lin2']['b'][None])


def _c_body(h_ref, i1_ref, i2_ref, p1_ref, p2_ref, c1_ref, c2_ref):
    i = pl.program_id(0)
    h = h_ref[...]
    ones = jnp.ones((BLK, 128), jnp.float32)
    for idx_ref, p_ref, c_ref in ((i1_ref, p1_ref, c1_ref),
                                  (i2_ref, p2_ref, c2_ref)):
        ids = idx_ref[0, 0, :]
        sel = (ids[:, None] ==
               jax.lax.broadcasted_iota(jnp.int32, (BLK, G), 1))
        sf = jnp.where(sel, 1.0, 0.0).astype(jnp.float32)
        ps = jax.lax.dot_general(sf, h, (((0,), (0,)), ((), ())),
                                 preferred_element_type=jnp.float32)
        cs = jax.lax.dot_general(sf, ones, (((0,), (0,)), ((), ())),
                                 preferred_element_type=jnp.float32)

        @pl.when(i == 0)
        def _():
            p_ref[...] = ps
            c_ref[...] = cs

        @pl.when(i > 0)
        def _():
            p_ref[...] += ps
            c_ref[...] += cs


def _pool(h, idx1, idx2):
    idx1 = idx1.astype(jnp.int32).reshape(NB, 1, BLK)
    idx2 = idx2.astype(jnp.int32).reshape(NB, 1, BLK)
    return pl.pallas_call(
        _c_body,
        grid=(NB,),
        in_specs=[
            pl.BlockSpec((BLK, 256), lambda i: (i, 0)),
            pl.BlockSpec((1, 1, BLK), lambda i: (i, 0, 0)),
            pl.BlockSpec((1, 1, BLK), lambda i: (i, 0, 0)),
        ],
        out_specs=[
            pl.BlockSpec((G, 256), lambda i: (0, 0)),
            pl.BlockSpec((G, 256), lambda i: (0, 0)),
            pl.BlockSpec((G, 128), lambda i: (0, 0)),
            pl.BlockSpec((G, 128), lambda i: (0, 0)),
        ],
        out_shape=[
            jax.ShapeDtypeStruct((G, 256), jnp.float32),
            jax.ShapeDtypeStruct((G, 256), jnp.float32),
            jax.ShapeDtypeStruct((G, 128), jnp.float32),
            jax.ShapeDtypeStruct((G, 128), jnp.float32),
        ],
    )(h, idx1, idx2)


def _d_body(p1_ref, p2_ref, c1_ref, c2_ref, wma_ref, wmb_ref, bm_ref,
            wm2_ref, bm2_ref, wl1_ref, bl1_ref, wl2_ref, bl2_ref, o_ref):
    x1 = p1_ref[...] / jnp.clip(c1_ref[...], 1.0, None)[:, 0:1]
    x2 = p2_ref[...] / jnp.clip(c2_ref[...], 1.0, None)[:, 0:1]
    t = jax.nn.relu(
        jnp.dot(x1, wma_ref[...], preferred_element_type=jnp.float32)
        + jnp.dot(x2, wmb_ref[...], preferred_element_type=jnp.float32)
        + bm_ref[...])
    t = jnp.dot(t, wm2_ref[...], preferred_element_type=jnp.float32) + bm2_ref[...]
    t = jax.nn.relu(jnp.dot(t, wl1_ref[...],
                            preferred_element_type=jnp.float32) + bl1_ref[...])
    o_ref[...] = jnp.dot(t, wl2_ref[...],
                         preferred_element_type=jnp.float32) + bl2_ref[...]


def _head(p1, p2, c1, c2, mlp, lin1, lin2):
    wm = mlp['lin1']['W']
    wl2 = jnp.pad(lin2['W'], ((0, 0), (0, 7)))
    bl2 = jnp.pad(lin2['b'], (0, 7))
    return pl.pallas_call(
        _d_body,
        out_shape=jax.ShapeDtypeStruct((G, 128), jnp.float32),
    )(p1, p2, c1, c2, wm[:256], wm[256:], mlp['lin1']['b'][None],
      mlp['lin2']['W'], mlp['lin2']['b'][None],
      lin1['W'], lin1['b'][None], wl2, bl2[None])[:, :121]


def kernel(x, edge_index_1, edge_index_2, index_1, index_2, params):
    zeros = jnp.zeros((NROW, CW), jnp.float32)
    src1, dst1, rounds = _pad_edges(edge_index_1[0], edge_index_1[1])
    src2, dst2, _ = _pad_edges(edge_index_2[0], edge_index_2[1])

    def layer(h, p1, p2, mlp, w1p=None):
        agg1 = _sc_segsum(h, src1, dst1, rounds, zeros)
        agg2 = _sc_segsum(h, src2, dst2, rounds, zeros)
        if w1p is not None:
            p1 = {**p1, 'lin1': {'W': w1p[0], 'b': p1['lin1']['b']}}
            p2 = {**p2, 'lin1': {'W': w1p[1], 'b': p2['lin1']['b']}}
        h1, h2, s1, s2 = _conv_pair(h, agg1, agg2, p1, p2)
        return _bn_mlp(h1, h2, s1, s2, p1, p2, mlp)

    xp = jnp.pad(x, ((0, 0), (0, 128 - x.shape[1])))
    w1p = (jnp.pad(params['conv_1_1']['lin1']['W'], ((0, 26), (0, 0))),
           jnp.pad(params['conv_1_2']['lin1']['W'], ((0, 26), (0, 0))))
    h = layer(xp, params['conv_1_1'], params['conv_1_2'], params['mlp_1'], w1p)
    h = layer(h, params['conv_2_1'], params['conv_2_2'], params['mlp_2'])
    h = layer(h, params['conv_3_1'], params['conv_3_2'], params['mlp_2'])
    p1, p2, c1, c2 = _pool(h, index_1, index_2)
    return _head(p1, p2, c1, c2, params['mlp'], params['lin1'], params['lin2'])


# speedup vs baseline: 1.0817x; 1.0817x over previous
import functools

import jax
import jax.numpy as jnp
from jax import lax
from jax.experimental import pallas as pl
from jax.experimental.pallas import tpu as pltpu
from jax.experimental.pallas import tpu_sc as plsc

N = 50000
E = 800000
NTILE = 16
NROW = 3128
N_PAD = NROW * NTILE
CW = 32
CHUNK = 256
SUB = 128
NSUB = CHUNK // SUB


SROUND = 8


def _pad_edges(src, dst):
    per_sr = CHUNK * SROUND
    per_tile = -(-E // (NTILE * per_sr)) * per_sr
    e_pad = per_tile * NTILE
    pad = e_pad - E
    pad_src = (jnp.arange(pad, dtype=jnp.int32) * 97) % N
    pad_dst = jnp.arange(pad, dtype=jnp.int32) % (N_PAD - N) + N
    src = jnp.concatenate([src.astype(jnp.int32), pad_src]).reshape(e_pad // SUB, SUB)
    dst = jnp.concatenate([dst.astype(jnp.int32), pad_dst]).reshape(e_pad // SUB, SUB)
    return src, dst, per_tile // CHUNK


def _make_segsum(n_groups, rounds):
    n_pass = n_groups // 2
    mesh = plsc.VectorSubcoreMesh(core_axis_name="c", subcore_axis_name="s")

    def body(*refs):
        tables = refs[:n_groups]
        (src_hbm, dst_hbm, z_hbm, agg_hbm,
         src_v, dst_v, buf0, buf1, acc_sh,
         sg0, sg1, ss0, ss1) = refs[n_groups:]
        buf = (buf0, buf1)
        sg = (sg0, sg1)
        ss = (ss0, ss1)
        c = lax.axis_index("c")
        s = lax.axis_index("s")
        idxr = SROUND * NSUB
        nsr = rounds // SROUND

        for p in range(n_pass):
            for cv in range(2):
                g = 2 * p + cv
                tab = tables[g]

                @pl.when(c == cv)
                def _(tab=tab, g=g):
                    pltpu.sync_copy(z_hbm, acc_sh.at[pl.ds(s * NROW, NROW)])
                    plsc.subcore_barrier()

                    def fire_gather(b, r):
                        for j in range(NSUB):
                            pltpu.async_copy(
                                tab.at[src_v.at[r * NSUB + j]],
                                buf[b].at[pl.ds(j * SUB, SUB)], sg[b])

                    def fire_scatter(b, r):
                        for j in range(NSUB):
                            pltpu.async_copy(
                                buf[b].at[pl.ds(j * SUB, SUB)],
                                acc_sh.at[dst_v.at[r * NSUB + j]], ss[b],
                                add=True)

                    def drain(sem, b):
                        pltpu.make_async_copy(
                            tab.at[pl.ds(0, CHUNK)], buf[b], sem).wait()

                    def sround(k, _):
                        r0 = s * (rounds * NSUB) + k * idxr
                        pltpu.sync_copy(src_hbm.at[pl.ds(r0, idxr)], src_v)
                        pltpu.sync_copy(dst_hbm.at[pl.ds(r0, idxr)], dst_v)
                        fire_gather(0, 0)
                        for r in range(1, SROUND):
                            b = r % 2
                            if r >= 2:
                                drain(ss[b], b)
                            fire_gather(b, r)
                            drain(sg[1 - b], 1 - b)
                            fire_scatter(1 - b, r - 1)
                        last = (SROUND - 1) % 2
                        drain(sg[last], last)
                        fire_scatter(last, SROUND - 1)
                        drain(ss[0], 0)
                        drain(ss[1], 1)
                        return 0

                    lax.fori_loop(0, nsr, sround, 0)
                    plsc.subcore_barrier()
                    pltpu.sync_copy(
                        acc_sh.at[pl.ds(s * NROW, NROW)],
                        agg_hbm.at[g].at[pl.ds(s * NROW, NROW)])
                    plsc.subcore_barrier()

    return pl.kernel(
        body,
        mesh=mesh,
        compiler_params=pltpu.CompilerParams(use_tc_tiling_on_sc=False),
        out_type=jax.ShapeDtypeStruct((n_groups, N_PAD, CW), jnp.float32),
        scratch_types=[
            pltpu.VMEM((SROUND * NSUB, SUB), jnp.int32),
            pltpu.VMEM((SROUND * NSUB, SUB), jnp.int32),
            pltpu.VMEM((CHUNK, CW), jnp.float32),
            pltpu.VMEM((CHUNK, CW), jnp.float32),
            pltpu.VMEM_SHARED((N_PAD, CW), jnp.float32),
        ] + [pltpu.SemaphoreType.DMA] * 4,
    )


def _sc_segsum(x, src2, dst2, rounds, zeros):
    c_dim = x.shape[1]
    n_groups = c_dim // CW
    tables = [x[:, g * CW:(g + 1) * CW] for g in range(n_groups)]
    fn = _make_segsum(n_groups, rounds)
    agg = fn(*tables, src2, dst2, zeros)
    return jnp.moveaxis(agg[:, :N, :], 0, 1).reshape(N, c_dim)


BLK = 2000
NB = N // BLK
G = 512


def _a_body(x_ref, a1_ref, a2_ref, w1_ref, w2_ref, b1_ref, b2_ref,
            h1_ref, h2_ref, s1_ref, s2_ref):
    i = pl.program_id(0)
    for a_ref, w_ref, b_ref, h_ref, s_ref in (
            (a1_ref, w1_ref, b1_ref, h1_ref, s1_ref),
            (a2_ref, w2_ref, b2_ref, h2_ref, s2_ref)):
        h = jnp.dot(x_ref[...] + a_ref[...], w_ref[...],
                    preferred_element_type=jnp.float32) + b_ref[...]
        h_ref[...] = h
        z = jnp.zeros((6, 256), jnp.float32)
        ps = jnp.concatenate(
            [jnp.sum(h, 0)[None], jnp.sum(h * h, 0)[None], z], axis=0)

        @pl.when(i == 0)
        def _():
            s_ref[...] = ps

        @pl.when(i > 0)
        def _():
            s_ref[...] += ps


def _conv_pair(x, agg1, agg2, p1, p2):
    din = x.shape[1]
    return pl.pallas_call(
        _a_body,
        grid=(NB,),
        in_specs=[
            pl.BlockSpec((BLK, din), lambda i: (i, 0)),
            pl.BlockSpec((BLK, din), lambda i: (i, 0)),
            pl.BlockSpec((BLK, din), lambda i: (i, 0)),
            pl.BlockSpec((din, 256), lambda i: (0, 0)),
            pl.BlockSpec((din, 256), lambda i: (0, 0)),
            pl.BlockSpec((1, 256), lambda i: (0, 0)),
            pl.BlockSpec((1, 256), lambda i: (0, 0)),
        ],
        out_specs=[
            pl.BlockSpec((BLK, 256), lambda i: (i, 0)),
            pl.BlockSpec((BLK, 256), lambda i: (i, 0)),
            pl.BlockSpec((8, 256), lambda i: (0, 0)),
            pl.BlockSpec((8, 256), lambda i: (0, 0)),
        ],
        out_shape=[
            jax.ShapeDtypeStruct((N, 256), jnp.float32),
            jax.ShapeDtypeStruct((N, 256), jnp.float32),
            jax.ShapeDtypeStruct((8, 256), jnp.float32),
            jax.ShapeDtypeStruct((8, 256), jnp.float32),
        ],
    )(x, agg1, agg2, p1['lin1']['W'], p2['lin1']['W'],
      p1['lin1']['b'][None], p2['lin1']['b'][None])


def _b_body(h1_ref, h2_ref, s1_ref, s2_ref, g1_ref, be1_ref, g2_ref, be2_ref,
            w21_ref, c21_ref, w22_ref, c22_ref, wma_ref, wmb_ref, bm1_ref,
            wm2_ref, bm2_ref, o_ref):
    ys = []
    for h_ref, s_ref, g_ref, be_ref, w_ref, c_ref in (
            (h1_ref, s1_ref, g1_ref, be1_ref, w21_ref, c21_ref),
            (h2_ref, s2_ref, g2_ref, be2_ref, w22_ref, c22_ref)):
        mu = s_ref[0:1, :] / N
        var = s_ref[1:2, :] / N - mu * mu
        inv = jax.lax.rsqrt(var + 1e-5) * g_ref[...]
        hn = jax.nn.relu((h_ref[...] - mu) * inv + be_ref[...])
        ys.append(jax.nn.relu(
            jnp.dot(hn, w_ref[...], preferred_element_type=jnp.float32)
            + c_ref[...]))
    t = jax.nn.relu(
        jnp.dot(ys[0], wma_ref[...], preferred_element_type=jnp.float32)
        + jnp.dot(ys[1], wmb_ref[...], preferred_element_type=jnp.float32)
        + bm1_ref[...])
    o_ref[...] = jnp.dot(t, wm2_ref[...],
                         preferred_element_type=jnp.float32) + bm2_ref[...]


def _bn_mlp(h1, h2, s1, s2, p1, p2, mlp):
    row = pl.BlockSpec((BLK, 256), lambda i: (i, 0))
    w = pl.BlockSpec((256, 256), lambda i: (0, 0))
    b = pl.BlockSpec((1, 256), lambda i: (0, 0))
    s = pl.BlockSpec((8, 256), lambda i: (0, 0))
    wm = mlp['lin1']['W']
    return pl.pallas_call(
        _b_body,
        grid=(NB,),
        in_specs=[row, row, s, s, b, b, b, b, w, b, w, b, w, w, b, w, b],
        out_specs=row,
        out_shape=jax.ShapeDtypeStruct((N, 256), jnp.float32),
    )(h1, h2, s1, s2,
      p1['gamma'][None], p1['beta'][None], p2['gamma'][None], p2['beta'][None],
      p1['lin2']['W'], p1['lin2']['b'][None],
      p2['lin2']['W'], p2['lin2']['b'][None],
      wm[:256], wm[256:], mlp['lin1']['b'][None],
      mlp['lin2']['W'], mlp['lin2']['b'][None])


def _c_body(h_ref, i1_ref, i2_ref, p1_ref, p2_ref, c1_ref, c2_ref):
    i = pl.program_id(0)
    h = h_ref[...]
    ones = jnp.ones((BLK, 128), jnp.float32)
    for idx_ref, p_ref, c_ref in ((i1_ref, p1_ref, c1_ref),
                                  (i2_ref, p2_ref, c2_ref)):
        ids = idx_ref[0, 0, :]
        sel = (ids[:, None] ==
               jax.lax.broadcasted_iota(jnp.int32, (BLK, G), 1))
        sf = jnp.where(sel, 1.0, 0.0).astype(jnp.float32)
        ps = jax.lax.dot_general(sf, h, (((0,), (0,)), ((), ())),
                                 preferred_element_type=jnp.float32)
        cs = jax.lax.dot_general(sf, ones, (((0,), (0,)), ((), ())),
                                 preferred_element_type=jnp.float32)

        @pl.when(i == 0)
        def _():
            p_ref[...] = ps
            c_ref[...] = cs

        @pl.when(i > 0)
        def _():
            p_ref[...] += ps
            c_ref[...] += cs


def _pool(h, idx1, idx2):
    idx1 = idx1.astype(jnp.int32).reshape(NB, 1, BLK)
    idx2 = idx2.astype(jnp.int32).reshape(NB, 1, BLK)
    return pl.pallas_call(
        _c_body,
        grid=(NB,),
        in_specs=[
            pl.BlockSpec((BLK, 256), lambda i: (i, 0)),
            pl.BlockSpec((1, 1, BLK), lambda i: (i, 0, 0)),
            pl.BlockSpec((1, 1, BLK), lambda i: (i, 0, 0)),
        ],
        out_specs=[
            pl.BlockSpec((G, 256), lambda i: (0, 0)),
            pl.BlockSpec((G, 256), lambda i: (0, 0)),
            pl.BlockSpec((G, 128), lambda i: (0, 0)),
            pl.BlockSpec((G, 128), lambda i: (0, 0)),
        ],
        out_shape=[
            jax.ShapeDtypeStruct((G, 256), jnp.float32),
            jax.ShapeDtypeStruct((G, 256), jnp.float32),
            jax.ShapeDtypeStruct((G, 128), jnp.float32),
            jax.ShapeDtypeStruct((G, 128), jnp.float32),
        ],
    )(h, idx1, idx2)


def _d_body(p1_ref, p2_ref, c1_ref, c2_ref, wma_ref, wmb_ref, bm_ref,
            wm2_ref, bm2_ref, wl1_ref, bl1_ref, wl2_ref, bl2_ref, o_ref):
    x1 = p1_ref[...] / jnp.clip(c1_ref[...], 1.0, None)[:, 0:1]
    x2 = p2_ref[...] / jnp.clip(c2_ref[...], 1.0, None)[:, 0:1]
    t = jax.nn.relu(
        jnp.dot(x1, wma_ref[...], preferred_element_type=jnp.float32)
        + jnp.dot(x2, wmb_ref[...], preferred_element_type=jnp.float32)
        + bm_ref[...])
    t = jnp.dot(t, wm2_ref[...], preferred_element_type=jnp.float32) + bm2_ref[...]
    t = jax.nn.relu(jnp.dot(t, wl1_ref[...],
                            preferred_element_type=jnp.float32) + bl1_ref[...])
    o_ref[...] = jnp.dot(t, wl2_ref[...],
                         preferred_element_type=jnp.float32) + bl2_ref[...]


def _head(p1, p2, c1, c2, mlp, lin1, lin2):
    wm = mlp['lin1']['W']
    wl2 = jnp.pad(lin2['W'], ((0, 0), (0, 7)))
    bl2 = jnp.pad(lin2['b'], (0, 7))
    return pl.pallas_call(
        _d_body,
        out_shape=jax.ShapeDtypeStruct((G, 128), jnp.float32),
    )(p1, p2, c1, c2, wm[:256], wm[256:], mlp['lin1']['b'][None],
      mlp['lin2']['W'], mlp['lin2']['b'][None],
      lin1['W'], lin1['b'][None], wl2, bl2[None])[:, :121]


def kernel(x, edge_index_1, edge_index_2, index_1, index_2, params):
    zeros = jnp.zeros((NROW, CW), jnp.float32)
    src1, dst1, rounds = _pad_edges(edge_index_1[0], edge_index_1[1])
    src2, dst2, _ = _pad_edges(edge_index_2[0], edge_index_2[1])

    def layer(h, p1, p2, mlp, w1p=None):
        agg1 = _sc_segsum(h, src1, dst1, rounds, zeros)
        agg2 = _sc_segsum(h, src2, dst2, rounds, zeros)
        if w1p is not None:
            p1 = {**p1, 'lin1': {'W': w1p[0], 'b': p1['lin1']['b']}}
            p2 = {**p2, 'lin1': {'W': w1p[1], 'b': p2['lin1']['b']}}
        h1, h2, s1, s2 = _conv_pair(h, agg1, agg2, p1, p2)
        return _bn_mlp(h1, h2, s1, s2, p1, p2, mlp)

    xp = jnp.pad(x, ((0, 0), (0, 128 - x.shape[1])))
    w1p = (jnp.pad(params['conv_1_1']['lin1']['W'], ((0, 26), (0, 0))),
           jnp.pad(params['conv_1_2']['lin1']['W'], ((0, 26), (0, 0))))
    h = layer(xp, params['conv_1_1'], params['conv_1_2'], params['mlp_1'], w1p)
    h = layer(h, params['conv_2_1'], params['conv_2_2'], params['mlp_2'])
    h = layer(h, params['conv_3_1'], params['conv_3_2'], params['mlp_2'])
    p1, p2, c1, c2 = _pool(h, index_1, index_2)
    return _head(p1, p2, c1, c2, params['mlp'], params['lin1'], params['lin2'])
